# baseline (speedup 1.0000x reference)
import functools

import jax
import jax.numpy as jnp
from jax import lax
from jax.experimental import pallas as pl
from jax.experimental.pallas import tpu as pltpu
from jax.experimental.pallas import tpu_sc as plsc

N = 10000
NP = 10240
D = 256
DH = 128
NC = 2
NS = 16
CH = 64
CPB = 18
NGB = 3
NSB = 2
ROWS_PER_SUB = NP // NS
ZROWS = 64

_mesh = plsc.VectorSubcoreMesh(
    core_axis_name="c", subcore_axis_name="s", num_cores=NC, num_subcores=NS)


@functools.lru_cache(maxsize=None)
def _make_spmm(nblocks):
    @functools.partial(
        pl.kernel,
        out_type=(jax.ShapeDtypeStruct((NP, DH), jnp.float32),
                  jax.ShapeDtypeStruct((NP, DH), jnp.float32)),
        mesh=_mesh,
        scratch_types=(
            [pltpu.VMEM((CPB, CH), jnp.int32)] * 2 +
            [pltpu.VMEM((CPB * CH,), jnp.float32)] +
            [pltpu.VMEM((CH, DH), jnp.float32)] * (NGB + NSB) +
            [pltpu.VMEM_SHARED((NP, DH), jnp.float32)] +
            [pltpu.SemaphoreType.DMA] * (NGB + NSB)
        ),
    )
    def spmm(src_lo, src_hi, rdata, cdata, wdata, out_lo, out_hi,
             rbuf, cbuf, wbuf, g0, g1, g2, s0, s1, acc,
             gm0, gm1, gm2, sm0, sm1):
        c = lax.axis_index("c")
        s = lax.axis_index("s")
        gbufs, gsems = (g0, g1, g2), (gm0, gm1, gm2)
        sbufs, ssems = (s0, s1), (sm0, sm1)

        def zrow(r, carry):
            for jj in range(DH // 16):
                g0[r, pl.ds(jj * 16, 16)] = jnp.zeros((16,), jnp.float32)
            return carry
        lax.fori_loop(0, ZROWS, zrow, 0)
        for k in range(ROWS_PER_SUB // ZROWS):
            pltpu.sync_copy(
                g0, acc.at[pl.ds(s * ROWS_PER_SUB + k * ZROWS, ZROWS)])
        plsc.subcore_barrier()

        def run(src_hbm, out_hbm):
            def start_gather(gi, k):
                pltpu.async_copy(
                    src_hbm.at[cbuf.at[k]], gbufs[gi], gsems[gi])

            def wait_gather(gi, k):
                pltpu.make_async_copy(
                    src_hbm.at[cbuf.at[k]], gbufs[gi], gsems[gi]).wait()

            def start_scatter(si, k):
                pltpu.async_copy(
                    sbufs[si], acc.at[rbuf.at[k]], ssems[si], add=True)

            def wait_scatter(si, k):
                pltpu.make_async_copy(
                    sbufs[si], acc.at[rbuf.at[k]], ssems[si]).wait()

            def scale(si, gi, k):
                gref, sref = gbufs[gi], sbufs[si]

                @plsc.parallel_loop(0, CH // 16, unroll=2)
                def grp(g):
                    woff = pl.multiple_of(k * CH + g * 16, 16)
                    wv16 = wbuf[pl.ds(woff, 16)]
                    for l in range(16):
                        wb = lax.gather(
                            wv16, jnp.full((16, 1), l, jnp.int32),
                            lax.GatherDimensionNumbers(
                                offset_dims=(), collapsed_slice_dims=(0,),
                                start_index_map=(0,)),
                            (1,),
                            mode=lax.GatherScatterMode.PROMISE_IN_BOUNDS)
                        i = g * 16 + l
                        for jj in range(DH // 16):
                            sl = pl.ds(jj * 16, 16)
                            sref[i, sl] = gref[i, sl] * wb

            def block(bi, carry):
                pltpu.sync_copy(rdata.at[s, bi], rbuf)
                pltpu.sync_copy(cdata.at[s, bi], cbuf)
                pltpu.sync_copy(wdata.at[s, bi], wbuf)
                start_gather(0, 0)
                start_gather(1, 1)

                ngrp = CPB // 6

                def group(q, qcarry):
                    c6 = q * 6
                    for u in range(6):
                        ck = c6 + u
                        gi, si = u % NGB, u % NSB
                        if u < 4:
                            start_gather((u + 2) % NGB, ck + 2)
                        else:
                            @pl.when(q < ngrp - 1)
                            def _():
                                start_gather((u + 2) % NGB, ck + 2)
                        wait_gather(gi, ck)
                        if u >= 2:
                            wait_scatter(si, ck - 2)
                        else:
                            @pl.when(q > 0)
                            def _():
                                wait_scatter(si, ck - 2)
                        scale(si, gi, ck)
                        start_scatter(si, ck)
                    return qcarry
                lax.fori_loop(0, ngrp, group, 0)
                wait_scatter(CPB % NSB, CPB - 2)
                wait_scatter((CPB + 1) % NSB, CPB - 1)
                return carry
            lax.fori_loop(0, nblocks, block, 0)
            plsc.subcore_barrier()
            base = s * ROWS_PER_SUB
            pltpu.sync_copy(acc.at[pl.ds(base, ROWS_PER_SUB)],
                            out_hbm.at[pl.ds(base, ROWS_PER_SUB)])

        @pl.when(c == 0)
        def _():
            run(src_lo, out_lo)

        @pl.when(c == 1)
        def _():
            run(src_hi, out_hi)

    return spmm


BN = 400


def _dense_body(x_b, h1lo_b, h1hi_b, h2lo_b, h2hi_b,
                wx, w1lo, w1hi, w2lo, w2hi, b_b, out_b):
    acc = jnp.dot(x_b[...], wx[...], preferred_element_type=jnp.float32)
    acc += jnp.dot(h1lo_b[...], w1lo[...], preferred_element_type=jnp.float32)
    acc += jnp.dot(h1hi_b[...], w1hi[...], preferred_element_type=jnp.float32)
    acc += jnp.dot(h2lo_b[...], w2lo[...], preferred_element_type=jnp.float32)
    acc += jnp.dot(h2hi_b[...], w2hi[...], preferred_element_type=jnp.float32)
    out_b[...] = acc + b_b[...]


_dense = pl.pallas_call(
    _dense_body,
    grid=(N // BN,),
    in_specs=[
        pl.BlockSpec((BN, D), lambda i: (i, 0)),
        pl.BlockSpec((BN, DH), lambda i: (i, 0)),
        pl.BlockSpec((BN, DH), lambda i: (i, 0)),
        pl.BlockSpec((BN, DH), lambda i: (i, 0)),
        pl.BlockSpec((BN, DH), lambda i: (i, 0)),
        pl.BlockSpec((D, D), lambda i: (0, 0)),
        pl.BlockSpec((DH, D), lambda i: (0, 0)),
        pl.BlockSpec((DH, D), lambda i: (0, 0)),
        pl.BlockSpec((DH, D), lambda i: (0, 0)),
        pl.BlockSpec((DH, D), lambda i: (0, 0)),
        pl.BlockSpec((1, D), lambda i: (0, 0)),
    ],
    out_specs=pl.BlockSpec((BN, D), lambda i: (i, 0)),
    out_shape=jax.ShapeDtypeStruct((N, D), jnp.float32),
)


def kernel(x, edge_index, edge_weight, W, b):
    e = edge_index.shape[1]
    eb = NS * CPB * CH
    nblocks = -(-e // eb)
    ep = eb * nblocks
    rows = jnp.pad(edge_index[0], (0, ep - e))
    cols = jnp.pad(edge_index[1], (0, ep - e))
    w = jnp.pad(edge_weight, (0, ep - e))
    rdata = rows.reshape(NS, nblocks, CPB, CH)
    cdata = cols.reshape(NS, nblocks, CPB, CH)
    wdata = w.reshape(NS, nblocks, CPB * CH)

    x_lo = x[:, :DH]
    x_hi = x[:, DH:]
    spmm = _make_spmm(nblocks)
    h1_lo, h1_hi = spmm(x_lo, x_hi, rdata, cdata, wdata)
    h2_lo, h2_hi = spmm(h1_lo, h1_hi, rdata, cdata, wdata)

    wt = W.T
    out = _dense(x, h1_lo[:N], h1_hi[:N], h2_lo[:N], h2_hi[:N],
                 wt[:D], wt[D:D + DH], wt[D + DH:2 * D],
                 wt[2 * D:2 * D + DH], wt[2 * D + DH:],
                 b.reshape(1, D))
    return out


# speedup vs baseline: 1.2169x; 1.2169x over previous
import functools

import jax
import jax.numpy as jnp
from jax import lax
from jax.experimental import pallas as pl
from jax.experimental.pallas import tpu as pltpu
from jax.experimental.pallas import tpu_sc as plsc

N = 10000
NP = 10240
D = 256
DH = 128
NC = 2
NS = 16
CH = 128
SH = 64
CPB = 8
ROWS_PER_SUB = NP // NS
ZROWS = 128

_mesh = plsc.VectorSubcoreMesh(
    core_axis_name="c", subcore_axis_name="s", num_cores=NC, num_subcores=NS)


@functools.lru_cache(maxsize=None)
def _make_spmm(nblocks):
    @functools.partial(
        pl.kernel,
        out_type=(jax.ShapeDtypeStruct((NP, DH), jnp.float32),
                  jax.ShapeDtypeStruct((NP, DH), jnp.float32)),
        mesh=_mesh,
        scratch_types=(
            [pltpu.VMEM((CPB * 2, SH), jnp.int32)] +
            [pltpu.VMEM((CPB, CH), jnp.int32)] +
            [pltpu.VMEM((CPB * CH,), jnp.float32)] +
            [pltpu.VMEM((CH, DH), jnp.float32)] * 2 +
            [pltpu.VMEM((SH, DH), jnp.float32)] +
            [pltpu.VMEM_SHARED((NP, DH), jnp.float32)] +
            [pltpu.SemaphoreType.DMA] * 3
        ),
    )
    def spmm(src_lo, src_hi, rdata, cdata, wdata, out_lo, out_hi,
             rbuf, cbuf, wbuf, g0, g1, sbuf, acc,
             gm0, gm1, smm):
        c = lax.axis_index("c")
        s = lax.axis_index("s")
        gbufs, gsems = (g0, g1), (gm0, gm1)

        def zrow(r, carry):
            for jj in range(DH // 16):
                g0[r, pl.ds(jj * 16, 16)] = jnp.zeros((16,), jnp.float32)
            return carry
        lax.fori_loop(0, ZROWS, zrow, 0)
        for k in range(ROWS_PER_SUB // ZROWS):
            pltpu.sync_copy(
                g0, acc.at[pl.ds(s * ROWS_PER_SUB + k * ZROWS, ZROWS)])
        plsc.subcore_barrier()

        def run(src_hbm, out_hbm):
            def start_gather(gi, k):
                pltpu.async_copy(
                    src_hbm.at[cbuf.at[k]], gbufs[gi], gsems[gi])

            def wait_gather(gi, k):
                pltpu.make_async_copy(
                    src_hbm.at[cbuf.at[k]], gbufs[gi], gsems[gi]).wait()

            def start_scatter(k, h):
                pltpu.async_copy(
                    sbuf, acc.at[rbuf.at[k * 2 + h]], smm, add=True)

            def wait_scatter(k, h):
                pltpu.make_async_copy(
                    sbuf, acc.at[rbuf.at[k * 2 + h]], smm).wait()

            def scale_half(gi, k, h):
                gref = gbufs[gi]

                @plsc.parallel_loop(0, SH // 16)
                def grp(g):
                    woff = pl.multiple_of(k * CH + h * SH + g * 16, 16)
                    wv16 = wbuf[pl.ds(woff, 16)]
                    for l in range(16):
                        wb = lax.gather(
                            wv16, jnp.full((16, 1), l, jnp.int32),
                            lax.GatherDimensionNumbers(
                                offset_dims=(), collapsed_slice_dims=(0,),
                                start_index_map=(0,)),
                            (1,),
                            mode=lax.GatherScatterMode.PROMISE_IN_BOUNDS)
                        i = g * 16 + l
                        for jj in range(DH // 16):
                            sl = pl.ds(jj * 16, 16)
                            sbuf[i, sl] = gref[h * SH + i, sl] * wb

            def do_chunk(gi, k, first):
                wait_gather(gi, k)
                for h in range(2):
                    if first and h == 0:
                        pass
                    elif h == 0:
                        wait_scatter(k - 1, 1)
                    else:
                        wait_scatter(k, 0)
                    scale_half(gi, k, h)
                    start_scatter(k, h)

            def block(bi, carry):
                pltpu.sync_copy(rdata.at[s, bi], rbuf)
                pltpu.sync_copy(cdata.at[s, bi], cbuf)
                pltpu.sync_copy(wdata.at[s, bi], wbuf)
                start_gather(0, 0)

                def pair(p, pcarry):
                    k0 = p * 2
                    start_gather(1, k0 + 1)

                    @pl.when(p == 0)
                    def _():
                        do_chunk(0, k0, True)

                    @pl.when(p > 0)
                    def _():
                        do_chunk(0, k0, False)

                    @pl.when(p < CPB // 2 - 1)
                    def _():
                        start_gather(0, k0 + 2)
                    do_chunk(1, k0 + 1, False)
                    return pcarry
                lax.fori_loop(0, CPB // 2, pair, 0)
                wait_scatter(CPB - 1, 1)
                return carry
            lax.fori_loop(0, nblocks, block, 0)
            plsc.subcore_barrier()
            base = s * ROWS_PER_SUB
            pltpu.sync_copy(acc.at[pl.ds(base, ROWS_PER_SUB)],
                            out_hbm.at[pl.ds(base, ROWS_PER_SUB)])

        @pl.when(c == 0)
        def _():
            run(src_lo, out_lo)

        @pl.when(c == 1)
        def _():
            run(src_hi, out_hi)

    return spmm


BN = 400


def _dense_body(x_b, h1lo_b, h1hi_b, h2lo_b, h2hi_b,
                wx, w1lo, w1hi, w2lo, w2hi, b_b, out_b):
    acc = jnp.dot(x_b[...], wx[...], preferred_element_type=jnp.float32)
    acc += jnp.dot(h1lo_b[...], w1lo[...], preferred_element_type=jnp.float32)
    acc += jnp.dot(h1hi_b[...], w1hi[...], preferred_element_type=jnp.float32)
    acc += jnp.dot(h2lo_b[...], w2lo[...], preferred_element_type=jnp.float32)
    acc += jnp.dot(h2hi_b[...], w2hi[...], preferred_element_type=jnp.float32)
    out_b[...] = acc + b_b[...]


_dense = pl.pallas_call(
    _dense_body,
    grid=(N // BN,),
    in_specs=[
        pl.BlockSpec((BN, D), lambda i: (i, 0)),
        pl.BlockSpec((BN, DH), lambda i: (i, 0)),
        pl.BlockSpec((BN, DH), lambda i: (i, 0)),
        pl.BlockSpec((BN, DH), lambda i: (i, 0)),
        pl.BlockSpec((BN, DH), lambda i: (i, 0)),
        pl.BlockSpec((D, D), lambda i: (0, 0)),
        pl.BlockSpec((DH, D), lambda i: (0, 0)),
        pl.BlockSpec((DH, D), lambda i: (0, 0)),
        pl.BlockSpec((DH, D), lambda i: (0, 0)),
        pl.BlockSpec((DH, D), lambda i: (0, 0)),
        pl.BlockSpec((1, D), lambda i: (0, 0)),
    ],
    out_specs=pl.BlockSpec((BN, D), lambda i: (i, 0)),
    out_shape=jax.ShapeDtypeStruct((N, D), jnp.float32),
)


def kernel(x, edge_index, edge_weight, W, b):
    e = edge_index.shape[1]
    eb = NS * CPB * CH
    nblocks = -(-e // eb)
    ep = eb * nblocks
    rows = jnp.pad(edge_index[0], (0, ep - e))
    cols = jnp.pad(edge_index[1], (0, ep - e))
    w = jnp.pad(edge_weight, (0, ep - e))
    rdata = rows.reshape(NS, nblocks, CPB * 2, SH)
    cdata = cols.reshape(NS, nblocks, CPB, CH)
    wdata = w.reshape(NS, nblocks, CPB * CH)

    x_lo = x[:, :DH]
    x_hi = x[:, DH:]
    spmm = _make_spmm(nblocks)
    h1_lo, h1_hi = spmm(x_lo, x_hi, rdata, cdata, wdata)
    h2_lo, h2_hi = spmm(h1_lo, h1_hi, rdata, cdata, wdata)

    wt = W.T
    out = _dense(x, h1_lo[:N], h1_hi[:N], h2_lo[:N], h2_hi[:N],
                 wt[:D], wt[D:D + DH], wt[D + DH:2 * D],
                 wt[2 * D:2 * D + DH], wt[2 * D + DH:],
                 b.reshape(1, D))
    return out


# speedup vs baseline: 1.3631x; 1.1201x over previous
import functools

import jax
import jax.numpy as jnp
from jax import lax
from jax.experimental import pallas as pl
from jax.experimental.pallas import tpu as pltpu
from jax.experimental.pallas import tpu_sc as plsc

N = 10000
NP = 10240
D = 256
DH = 128
NC = 2
NS = 16
CH = 128
ROWS_PER_SUB = NP // NS
ZROWS = 128

_mesh = plsc.VectorSubcoreMesh(
    core_axis_name="c", subcore_axis_name="s", num_cores=NC, num_subcores=NS)


@functools.lru_cache(maxsize=None)
def _make_spmm(nchunks):
    @functools.partial(
        pl.kernel,
        out_type=(jax.ShapeDtypeStruct((NP, DH), jnp.float32),
                  jax.ShapeDtypeStruct((NP, DH), jnp.float32)),
        mesh=_mesh,
        scratch_types=[
            pltpu.VMEM((nchunks, CH), jnp.int32),
            pltpu.VMEM((nchunks, CH), jnp.int32),
            pltpu.VMEM((nchunks * CH,), jnp.float32),
            pltpu.VMEM((CH, DH), jnp.float32),
            pltpu.VMEM_SHARED((NP, DH), jnp.float32),
            pltpu.SemaphoreType.DMA,
        ],
    )
    def spmm(src_lo, src_hi, rows3, cols3, w2, out_lo, out_hi,
             rows_all, cols_all, w_all, gbuf, acc, gsem):
        c = lax.axis_index("c")
        s = lax.axis_index("s")

        pltpu.sync_copy(rows3.at[s], rows_all)
        pltpu.sync_copy(cols3.at[s], cols_all)
        pltpu.sync_copy(w2.at[s], w_all)

        def zrow(r, carry):
            for jj in range(DH // 16):
                gbuf[r, pl.ds(jj * 16, 16)] = jnp.zeros((16,), jnp.float32)
            return carry
        lax.fori_loop(0, ZROWS, zrow, 0)
        for k in range(ROWS_PER_SUB // ZROWS):
            pltpu.sync_copy(
                gbuf, acc.at[pl.ds(s * ROWS_PER_SUB + k * ZROWS, ZROWS)])
        plsc.subcore_barrier()

        def run(src_hbm, out_hbm):
            def chunk(j, carry):
                pltpu.async_copy(src_hbm.at[cols_all.at[j]], gbuf, gsem).wait()

                def scale(g, gcarry):
                    wv16 = w_all[pl.ds(j * CH + g * 16, 16)]
                    for l in range(16):
                        wb = lax.gather(
                            wv16, jnp.full((16, 1), l, jnp.int32),
                            lax.GatherDimensionNumbers(
                                offset_dims=(), collapsed_slice_dims=(0,),
                                start_index_map=(0,)),
                            (1,), mode=lax.GatherScatterMode.PROMISE_IN_BOUNDS)
                        i = g * 16 + l
                        for jj in range(DH // 16):
                            sl = pl.ds(jj * 16, 16)
                            gbuf[i, sl] = gbuf[i, sl] * wb
                    return gcarry
                lax.fori_loop(0, CH // 16, scale, 0)
                pltpu.sync_copy(gbuf, acc.at[rows_all.at[j]], add=True)
                return carry
            lax.fori_loop(0, nchunks, chunk, 0)
            plsc.subcore_barrier()
            base = s * ROWS_PER_SUB
            pltpu.sync_copy(acc.at[pl.ds(base, ROWS_PER_SUB)],
                            out_hbm.at[pl.ds(base, ROWS_PER_SUB)])

        @pl.when(c == 0)
        def _():
            run(src_lo, out_lo)

        @pl.when(c == 1)
        def _():
            run(src_hi, out_hi)

    return spmm


BN = 400


def _dense_a_body(x_b, h1lo_b, h1hi_b, wx, w1lo, w1hi, b_b, out_b):
    acc = jnp.dot(x_b[...], wx[...], preferred_element_type=jnp.float32)
    acc += jnp.dot(h1lo_b[...], w1lo[...], preferred_element_type=jnp.float32)
    acc += jnp.dot(h1hi_b[...], w1hi[...], preferred_element_type=jnp.float32)
    out_b[...] = acc + b_b[...]


_dense_a = pl.pallas_call(
    _dense_a_body,
    grid=(N // BN,),
    in_specs=[
        pl.BlockSpec((BN, D), lambda i: (i, 0)),
        pl.BlockSpec((BN, DH), lambda i: (i, 0)),
        pl.BlockSpec((BN, DH), lambda i: (i, 0)),
        pl.BlockSpec((D, D), lambda i: (0, 0)),
        pl.BlockSpec((DH, D), lambda i: (0, 0)),
        pl.BlockSpec((DH, D), lambda i: (0, 0)),
        pl.BlockSpec((1, D), lambda i: (0, 0)),
    ],
    out_specs=pl.BlockSpec((BN, D), lambda i: (i, 0)),
    out_shape=jax.ShapeDtypeStruct((N, D), jnp.float32),
)


def _dense_b_body(part_b, h2lo_b, h2hi_b, w2lo, w2hi, out_b):
    acc = part_b[...]
    acc += jnp.dot(h2lo_b[...], w2lo[...], preferred_element_type=jnp.float32)
    acc += jnp.dot(h2hi_b[...], w2hi[...], preferred_element_type=jnp.float32)
    out_b[...] = acc


_dense_b = pl.pallas_call(
    _dense_b_body,
    grid=(N // BN,),
    in_specs=[
        pl.BlockSpec((BN, D), lambda i: (i, 0)),
        pl.BlockSpec((BN, DH), lambda i: (i, 0)),
        pl.BlockSpec((BN, DH), lambda i: (i, 0)),
        pl.BlockSpec((DH, D), lambda i: (0, 0)),
        pl.BlockSpec((DH, D), lambda i: (0, 0)),
    ],
    out_specs=pl.BlockSpec((BN, D), lambda i: (i, 0)),
    out_shape=jax.ShapeDtypeStruct((N, D), jnp.float32),
)


def kernel(x, edge_index, edge_weight, W, b):
    e = edge_index.shape[1]
    nchunks = -(-e // (NS * CH))
    ep = NS * CH * nchunks
    rows = jnp.pad(edge_index[0], (0, ep - e))
    cols = jnp.pad(edge_index[1], (0, ep - e))
    w = jnp.pad(edge_weight, (0, ep - e))
    rows3 = rows.reshape(NS, nchunks, CH)
    cols3 = cols.reshape(NS, nchunks, CH)
    w2 = w.reshape(NS, nchunks * CH)

    x_lo = x[:, :DH]
    x_hi = x[:, DH:]
    spmm = _make_spmm(nchunks)
    h1_lo, h1_hi = spmm(x_lo, x_hi, rows3, cols3, w2)
    h2_lo, h2_hi = spmm(h1_lo, h1_hi, rows3, cols3, w2)

    wt = W.T
    part = _dense_a(x, h1_lo[:N], h1_hi[:N],
                    wt[:D], wt[D:D + DH], wt[D + DH:2 * D], b.reshape(1, D))
    out = _dense_b(part, h2_lo[:N], h2_hi[:N],
                   wt[2 * D:2 * D + DH], wt[2 * D + DH:])
    return out
